# baseline (speedup 1.0000x reference)
import functools

import jax
import jax.numpy as jnp
from jax import lax
from jax.experimental import pallas as pl
from jax.experimental.pallas import tpu as pltpu
from jax.experimental.pallas import tpu_sc as plsc

_B = 4096
_L = 200
_D = 128
_NW = 32
_SEQ_PER_W = _B // _NW


_NCH = _D // 16
_GROUP = 25


def _sc_body(x_hbm, tab_hbm, out_hbm, tab_v, buf, insem, outsem):
    wid = lax.axis_index("s") * 2 + lax.axis_index("c")
    base = wid * _SEQ_PER_W
    pltpu.sync_copy(tab_hbm, tab_v)
    t0 = [tab_v[0, pl.ds(16 * j, 16)] for j in range(_NCH)]

    def in_cp(s):
        return pltpu.make_async_copy(x_hbm.at[base + s], buf.at[s % 3], insem)

    def out_cp(s):
        return pltpu.make_async_copy(buf.at[s % 3], out_hbm.at[base + s], outsem)

    in_cp(0).start()
    in_cp(1).start()

    def seq_body(s, carry):
        p = s % 3
        in_cp(s).wait()
        bufp = buf.at[p]

        @plsc.parallel_loop(0, _L, 1, unroll=8)
        def _row(l):
            xs = [bufp[l, pl.ds(16 * j, 16)] for j in range(_NCH)]
            nz = xs[0] != 0.0
            for v in xs[1:]:
                nz = nz | (v != 0.0)
            cnt = plsc.all_reduce_population_count(nz)
            m = cnt > 0
            for j in range(_NCH):
                t = jnp.where(m, tab_v[l + 1, pl.ds(16 * j, 16)], t0[j])
                bufp[l, pl.ds(16 * j, 16)] = xs[j] + t
        out_cp(s).start()

        @pl.when(s >= 1)
        def _():
            out_cp(s - 1).wait()

        @pl.when(s + 2 < _SEQ_PER_W)
        def _():
            in_cp(s + 2).start()

        return carry

    lax.fori_loop(0, _SEQ_PER_W, seq_body, 0)
    out_cp(_SEQ_PER_W - 1).wait()


def kernel(x, pos_table):
    B, L, D = x.shape
    mesh = plsc.VectorSubcoreMesh(core_axis_name="c", subcore_axis_name="s")
    run = functools.partial(
        pl.kernel,
        mesh=mesh,
        compiler_params=pltpu.CompilerParams(needs_layout_passes=False),
        out_type=jax.ShapeDtypeStruct((B, L, D), jnp.float32),
        scratch_types=[
            pltpu.VMEM((L + 1, D), jnp.float32),
            pltpu.VMEM((3, L, D), jnp.float32),
            pltpu.SemaphoreType.DMA,
            pltpu.SemaphoreType.DMA,
        ],
    )(_sc_body)
    return run(x, pos_table)


# speedup vs baseline: 4.5580x; 4.5580x over previous
import functools

import jax
import jax.numpy as jnp
from jax import lax
from jax.experimental import pallas as pl
from jax.experimental.pallas import tpu as pltpu
from jax.experimental.pallas import tpu_sc as plsc

_B = 4096
_L = 200
_D = 128
_NW = 32
_SEQ_PER_W = _B // _NW


_NCH = _D // 16
_GROUP = 25


def _sc_body(x_hbm, tab_hbm, out_hbm, tab_v, buf, insem, outsem):
    wid = lax.axis_index("s") * 2 + lax.axis_index("c")
    base = wid * _SEQ_PER_W
    pltpu.sync_copy(tab_hbm, tab_v)
    t0 = [tab_v[0, pl.ds(16 * j, 16)] for j in range(_NCH)]

    def in_cp(s):
        return pltpu.make_async_copy(x_hbm.at[base + s], buf.at[s % 3], insem)

    def out_cp(s):
        return pltpu.make_async_copy(buf.at[s % 3], out_hbm.at[base + s], outsem)

    in_cp(0).start()
    in_cp(1).start()

    def seq_body(s, carry):
        p = s % 3
        in_cp(s).wait()
        bufp = buf.at[p]

        @plsc.parallel_loop(0, _L, 1, unroll=2)
        def _row(l):
            xs = [bufp[l, pl.ds(16 * j, 16)] for j in range(_NCH)]
            nz = xs[0] != 0.0
            for v in xs[1:]:
                nz = nz | (v != 0.0)
            cnt = plsc.all_reduce_population_count(nz)
            m = cnt > 0
            for j in range(_NCH):
                t = jnp.where(m, tab_v[l + 1, pl.ds(16 * j, 16)], t0[j])
                bufp[l, pl.ds(16 * j, 16)] = xs[j] + t
        out_cp(s).start()

        @pl.when(s >= 1)
        def _():
            out_cp(s - 1).wait()

        @pl.when(s + 2 < _SEQ_PER_W)
        def _():
            in_cp(s + 2).start()

        return carry

    lax.fori_loop(0, _SEQ_PER_W, seq_body, 0)
    out_cp(_SEQ_PER_W - 1).wait()


def kernel(x, pos_table):
    B, L, D = x.shape
    mesh = plsc.VectorSubcoreMesh(core_axis_name="c", subcore_axis_name="s")
    run = functools.partial(
        pl.kernel,
        mesh=mesh,
        compiler_params=pltpu.CompilerParams(needs_layout_passes=False),
        out_type=jax.ShapeDtypeStruct((B, L, D), jnp.float32),
        scratch_types=[
            pltpu.VMEM((L + 1, D), jnp.float32),
            pltpu.VMEM((3, L, D), jnp.float32),
            pltpu.SemaphoreType.DMA,
            pltpu.SemaphoreType.DMA,
        ],
    )(_sc_body)
    return run(x, pos_table)


# speedup vs baseline: 4.7428x; 1.0405x over previous
import functools

import jax
import jax.numpy as jnp
from jax import lax
from jax.experimental import pallas as pl
from jax.experimental.pallas import tpu as pltpu
from jax.experimental.pallas import tpu_sc as plsc

_B = 4096
_L = 200
_D = 128
_NW = 32
_SEQ_PER_W = _B // _NW


_NCH = _D // 16
_GROUP = 25


def _sc_body(x_hbm, tab_hbm, out_hbm, tab_v, buf, insem, outsem):
    wid = lax.axis_index("s") * 2 + lax.axis_index("c")
    base = wid * _SEQ_PER_W
    pltpu.sync_copy(tab_hbm, tab_v)
    t0 = [tab_v[0, pl.ds(16 * j, 16)] for j in range(_NCH)]

    def in_cp(s):
        return pltpu.make_async_copy(x_hbm.at[base + s], buf.at[s % 3], insem)

    def out_cp(s):
        return pltpu.make_async_copy(buf.at[s % 3], out_hbm.at[base + s], outsem)

    def in_cp_q(s, q):
        return pltpu.make_async_copy(x_hbm.at[base + s], buf.at[q], insem)

    def out_cp_q(s, q):
        return pltpu.make_async_copy(buf.at[q], out_hbm.at[base + s], outsem)

    def compute(q):
        bufp = buf.at[q]

        @plsc.parallel_loop(0, _L, 1, unroll=2)
        def _row(l):
            xs = [bufp[l, pl.ds(16 * j, 16)] for j in range(_NCH)]
            nz = xs[0] != 0.0
            for v in xs[1:]:
                nz = nz | (v != 0.0)
            cnt = plsc.all_reduce_population_count(nz)
            m = cnt > 0
            for j in range(_NCH):
                t = jnp.where(m, tab_v[l + 1, pl.ds(16 * j, 16)], t0[j])
                bufp[l, pl.ds(16 * j, 16)] = xs[j] + t

    in_cp(0).start()
    in_cp(1).start()

    def seq_body3(k, carry):
        for q in range(3):
            s = 3 * k + q
            in_cp_q(s, q).wait()
            compute(q)
            out_cp_q(s, q).start()
            if q == 0:
                @pl.when(k > 0)
                def _():
                    out_cp_q(s - 1, 2).wait()
            else:
                out_cp_q(s - 1, q - 1).wait()
            in_cp_q(s + 2, (q + 2) % 3).start()
        return carry

    lax.fori_loop(0, _SEQ_PER_W // 3, seq_body3, 0)
    for s, q in ((_SEQ_PER_W - 2, 0), (_SEQ_PER_W - 1, 1)):
        in_cp_q(s, q).wait()
        compute(q)
        out_cp_q(s, q).start()
        out_cp_q(s - 1, (q + 2) % 3).wait()
    out_cp(_SEQ_PER_W - 1).wait()


def kernel(x, pos_table):
    B, L, D = x.shape
    mesh = plsc.VectorSubcoreMesh(core_axis_name="c", subcore_axis_name="s")
    run = functools.partial(
        pl.kernel,
        mesh=mesh,
        compiler_params=pltpu.CompilerParams(needs_layout_passes=False),
        out_type=jax.ShapeDtypeStruct((B, L, D), jnp.float32),
        scratch_types=[
            pltpu.VMEM((L + 1, D), jnp.float32),
            pltpu.VMEM((3, L, D), jnp.float32),
            pltpu.SemaphoreType.DMA,
            pltpu.SemaphoreType.DMA,
        ],
    )(_sc_body)
    return run(x, pos_table)


# speedup vs baseline: 4.7507x; 1.0017x over previous
import functools

import jax
import jax.numpy as jnp
from jax import lax
from jax.experimental import pallas as pl
from jax.experimental.pallas import tpu as pltpu
from jax.experimental.pallas import tpu_sc as plsc

_B = 4096
_L = 200
_D = 128
_NW = 32
_SEQ_PER_W = _B // _NW


_NCH = _D // 16
_NS = 4


def _sc_body(x_hbm, tab_hbm, out_hbm, tab_v, buf, insem, outsem):
    wid = lax.axis_index("s") * 2 + lax.axis_index("c")
    base = wid * _SEQ_PER_W
    pltpu.sync_copy(tab_hbm, tab_v)
    t0 = [tab_v[0, pl.ds(16 * j, 16)] for j in range(_NCH)]

    def in_cp(s):
        return pltpu.make_async_copy(x_hbm.at[base + s], buf.at[s % _NS], insem)

    def out_cp(s):
        return pltpu.make_async_copy(buf.at[s % _NS], out_hbm.at[base + s], outsem)

    def in_cp_q(s, q):
        return pltpu.make_async_copy(x_hbm.at[base + s], buf.at[q], insem)

    def out_cp_q(s, q):
        return pltpu.make_async_copy(buf.at[q], out_hbm.at[base + s], outsem)

    def compute(q):
        bufp = buf.at[q]

        @plsc.parallel_loop(0, _L, 1, unroll=2)
        def _row(l):
            xs = [bufp[l, pl.ds(16 * j, 16)] for j in range(_NCH)]
            nz = xs[0] != 0.0
            for v in xs[1:]:
                nz = nz | (v != 0.0)
            cnt = plsc.all_reduce_population_count(nz)
            m = cnt > 0
            for j in range(_NCH):
                t = jnp.where(m, tab_v[l + 1, pl.ds(16 * j, 16)], t0[j])
                bufp[l, pl.ds(16 * j, 16)] = xs[j] + t

    in_cp(0).start()
    in_cp(1).start()
    in_cp(2).start()

    def seq_body4(k, carry):
        for q in range(_NS):
            s = _NS * k + q
            in_cp_q(s, q).wait()
            compute(q)
            out_cp_q(s, q).start()
            if q == 0:
                @pl.when(k > 0)
                def _():
                    out_cp_q(s - 1, _NS - 1).wait()
            else:
                out_cp_q(s - 1, q - 1).wait()

            @pl.when(s + _NS - 1 < _SEQ_PER_W)
            def _():
                in_cp_q(s + _NS - 1, (q + _NS - 1) % _NS).start()
        return carry

    lax.fori_loop(0, _SEQ_PER_W // _NS, seq_body4, 0)
    out_cp(_SEQ_PER_W - 1).wait()


def kernel(x, pos_table):
    B, L, D = x.shape
    mesh = plsc.VectorSubcoreMesh(core_axis_name="c", subcore_axis_name="s")
    run = functools.partial(
        pl.kernel,
        mesh=mesh,
        compiler_params=pltpu.CompilerParams(needs_layout_passes=False),
        out_type=jax.ShapeDtypeStruct((B, L, D), jnp.float32),
        scratch_types=[
            pltpu.VMEM((L + 1, D), jnp.float32),
            pltpu.VMEM((_NS, L, D), jnp.float32),
            pltpu.SemaphoreType.DMA,
            pltpu.SemaphoreType.DMA,
        ],
    )(_sc_body)
    return run(x, pos_table)
